# baseline (speedup 1.0000x reference)
import functools

import jax
import jax.numpy as jnp
from jax.experimental import pallas as pl
from jax.experimental.pallas import tpu as pltpu


def _router_block(x_hbm, w_ref, o_ref, xslots, sems, *, nslots, split):
    i = pl.program_id(0)
    nblk = pl.num_programs(0)
    blk = xslots.shape[1]
    sub = blk // split

    def start(b):
        slot = jax.lax.rem(b, nslots)
        for s in range(split):
            pltpu.make_async_copy(
                x_hbm.at[pl.ds(b * blk + s * sub, sub), :],
                xslots.at[slot, pl.ds(s * sub, sub), :],
                sems.at[slot, s],
            ).start()

    def wait(b):
        slot = jax.lax.rem(b, nslots)
        for s in range(split):
            pltpu.make_async_copy(
                x_hbm.at[pl.ds(b * blk + s * sub, sub), :],
                xslots.at[slot, pl.ds(s * sub, sub), :],
                sems.at[slot, s],
            ).wait()

    @pl.when(i == 0)
    def _():
        for b in range(nslots - 1):
            start(b)

    @pl.when(i + nslots - 1 < nblk)
    def _():
        start(i + nslots - 1)

    slot = jax.lax.rem(i, nslots)
    wait(i)

    logits = jax.lax.dot_general(
        xslots[slot],
        w_ref[...],
        dimension_numbers=(((1,), (1,)), ((), ())),
        preferred_element_type=jnp.float32,
    )
    m = jnp.max(logits, axis=-1, keepdims=True)
    e = jnp.exp(logits - m)
    o_ref[...] = e / jnp.sum(e, axis=-1, keepdims=True)


@functools.partial(jax.jit, static_argnames=("block", "nslots", "split"))
def _router(x2d, W, block: int, nslots: int, split: int):
    n_tokens, d = x2d.shape
    n_experts = W.shape[0]
    grid = (n_tokens // block,)
    body = functools.partial(_router_block, nslots=nslots, split=split)
    return pl.pallas_call(
        body,
        grid=grid,
        in_specs=[
            pl.BlockSpec(memory_space=pl.ANY),
            pl.BlockSpec((n_experts, d), lambda i: (0, 0)),
        ],
        out_specs=pl.BlockSpec((block, n_experts), lambda i: (i, 0)),
        out_shape=jax.ShapeDtypeStruct((n_tokens, n_experts), jnp.float32),
        scratch_shapes=[
            pltpu.VMEM((nslots, block, d), jnp.float32),
            pltpu.SemaphoreType.DMA((nslots, split)),
        ],
        compiler_params=pltpu.CompilerParams(
            dimension_semantics=("arbitrary",),
            vmem_limit_bytes=100 * 1024 * 1024,
        ),
    )(x2d, W)


def kernel(x, W):
    b, t, d = x.shape
    out = _router(x.reshape(b * t, d), W, block=256, nslots=8, split=1)
    return out.reshape(b, t, W.shape[0])


# speedup vs baseline: 1.0029x; 1.0029x over previous
import functools

import jax
import jax.numpy as jnp
from jax.experimental import pallas as pl
from jax.experimental.pallas import tpu as pltpu


def _router_block(x_hbm, w_ref, o_ref, xslots, sems, *, nslots, split):
    i = pl.program_id(0)
    nblk = pl.num_programs(0)
    blk = xslots.shape[1]
    sub = blk // split

    def start(b):
        slot = jax.lax.rem(b, nslots)
        for s in range(split):
            pltpu.make_async_copy(
                x_hbm.at[pl.ds(b * blk + s * sub, sub), :],
                xslots.at[slot, pl.ds(s * sub, sub), :],
                sems.at[slot, s],
            ).start()

    def wait(b):
        slot = jax.lax.rem(b, nslots)
        for s in range(split):
            pltpu.make_async_copy(
                x_hbm.at[pl.ds(b * blk + s * sub, sub), :],
                xslots.at[slot, pl.ds(s * sub, sub), :],
                sems.at[slot, s],
            ).wait()

    @pl.when(i == 0)
    def _():
        for b in range(nslots - 1):
            start(b)

    @pl.when(i + nslots - 1 < nblk)
    def _():
        start(i + nslots - 1)

    slot = jax.lax.rem(i, nslots)
    wait(i)

    logits = jax.lax.dot_general(
        xslots[slot].astype(jnp.bfloat16),
        w_ref[...].astype(jnp.bfloat16),
        dimension_numbers=(((1,), (1,)), ((), ())),
        preferred_element_type=jnp.float32,
    )
    m = jnp.max(logits, axis=-1, keepdims=True)
    e = jnp.exp(logits - m)
    o_ref[...] = e / jnp.sum(e, axis=-1, keepdims=True)


@functools.partial(jax.jit, static_argnames=("block", "nslots", "split"))
def _router(x2d, W, block: int, nslots: int, split: int):
    n_tokens, d = x2d.shape
    n_experts = W.shape[0]
    grid = (n_tokens // block,)
    body = functools.partial(_router_block, nslots=nslots, split=split)
    return pl.pallas_call(
        body,
        grid=grid,
        in_specs=[
            pl.BlockSpec(memory_space=pl.ANY),
            pl.BlockSpec((n_experts, d), lambda i: (0, 0)),
        ],
        out_specs=pl.BlockSpec((block, n_experts), lambda i: (i, 0)),
        out_shape=jax.ShapeDtypeStruct((n_tokens, n_experts), jnp.float32),
        scratch_shapes=[
            pltpu.VMEM((nslots, block, d), jnp.float32),
            pltpu.SemaphoreType.DMA((nslots, split)),
        ],
        compiler_params=pltpu.CompilerParams(
            dimension_semantics=("arbitrary",),
            vmem_limit_bytes=100 * 1024 * 1024,
        ),
    )(x2d, W)


def kernel(x, W):
    b, t, d = x.shape
    out = _router(x.reshape(b * t, d), W, block=256, nslots=8, split=1)
    return out.reshape(b, t, W.shape[0])


# speedup vs baseline: 1.0173x; 1.0144x over previous
import functools

import jax
import jax.numpy as jnp
from jax.experimental import pallas as pl
from jax.experimental.pallas import tpu as pltpu


def _router_block(x_hbm, w_ref, o_ref, xslots, sems, *, nslots):
    i = pl.program_id(0)
    nblk = pl.num_programs(0)
    blk = xslots.shape[1]
    tblocks = x_hbm.shape[1] // blk

    def copy(b):
        slot = jax.lax.rem(b, nslots)
        return pltpu.make_async_copy(
            x_hbm.at[b // tblocks, pl.ds(jax.lax.rem(b, tblocks) * blk, blk), :],
            xslots.at[slot],
            sems.at[slot],
        )

    @pl.when(i == 0)
    def _():
        for b in range(nslots - 1):
            copy(b).start()

    @pl.when(i + nslots - 1 < nblk)
    def _():
        copy(i + nslots - 1).start()

    slot = jax.lax.rem(i, nslots)
    copy(i).wait()

    logits = jax.lax.dot_general(
        xslots[slot],
        w_ref[...],
        dimension_numbers=(((1,), (1,)), ((), ())),
        preferred_element_type=jnp.float32,
    )
    m = jnp.max(logits, axis=-1, keepdims=True)
    e = jnp.exp(logits - m)
    o_ref[0] = e / jnp.sum(e, axis=-1, keepdims=True)


@functools.partial(jax.jit, static_argnames=("block", "nslots"))
def _router(x, W, block: int, nslots: int):
    b, t, d = x.shape
    n_experts = W.shape[0]
    tblocks = t // block
    grid = (b * tblocks,)
    body = functools.partial(_router_block, nslots=nslots)
    return pl.pallas_call(
        body,
        grid=grid,
        in_specs=[
            pl.BlockSpec(memory_space=pl.ANY),
            pl.BlockSpec((n_experts, d), lambda i: (0, 0)),
        ],
        out_specs=pl.BlockSpec(
            (1, block, n_experts), lambda i: (i // tblocks, i % tblocks, 0)
        ),
        out_shape=jax.ShapeDtypeStruct((b, t, n_experts), jnp.float32),
        scratch_shapes=[
            pltpu.VMEM((nslots, block, d), jnp.float32),
            pltpu.SemaphoreType.DMA((nslots,)),
        ],
        compiler_params=pltpu.CompilerParams(
            dimension_semantics=("arbitrary",),
            vmem_limit_bytes=100 * 1024 * 1024,
        ),
    )(x, W)


def kernel(x, W):
    return _router(x, W, block=512, nslots=4)
